# baseline (speedup 1.0000x reference)
import functools

import jax
import jax.numpy as jnp
from jax import lax
from jax.experimental import pallas as pl
from jax.experimental.pallas import tpu as pltpu
from jax.experimental.pallas import tpu_sc as plsc

N = 10000
E = 320000
D = 128
DH = D // 2
T = 4
L = 16
NC = 2
NS = 16
NW = NC * NS
EPW = E // NW
CW = 80
NCH = EPW // CW
NPAD = 10240
CNT = NPAD * T
CNT_PW = CNT // NS
RPW = NPAD // NS
ZR = 160
BR = 1000

_mesh = plsc.VectorSubcoreMesh(
    core_axis_name="c", subcore_axis_name="s", num_cores=NC, num_subcores=NS
)


def _wid():
    return lax.axis_index("s") * NC + lax.axis_index("c")


@functools.partial(
    pl.kernel,
    out_type=jax.ShapeDtypeStruct((NC, CNT), jnp.float32),
    mesh=_mesh,
    compiler_params=pltpu.CompilerParams(needs_layout_passes=False),
    scratch_types=[
        pltpu.VMEM((N,), jnp.int32),
        pltpu.VMEM((NCH, CW), jnp.int32),
        pltpu.VMEM((NCH, CW), jnp.int32),
        pltpu.VMEM((NCH, CW), jnp.int32),
        pltpu.VMEM((CW,), jnp.float32),
        pltpu.VMEM((CNT_PW,), jnp.float32),
        pltpu.VMEM_SHARED((CNT,), jnp.float32),
    ],
)
def _sc_counts(src_hbm, tgt_hbm, nt_hbm, out_hbm,
               nt_v, src_v, tgt_v, seg_v, ones_v, zero_v, cnt_sh):
    cid = lax.axis_index("c")
    sid = lax.axis_index("s")
    wid = _wid()

    @pl.loop(0, CW // L)
    def _(i):
        ones_v[pl.ds(i * L, L)] = jnp.ones((L,), jnp.float32)

    @pl.loop(0, CNT_PW // L)
    def _(i):
        zero_v[pl.ds(i * L, L)] = jnp.zeros((L,), jnp.float32)

    pltpu.sync_copy(zero_v, cnt_sh.at[pl.ds(sid * CNT_PW, CNT_PW)])
    pltpu.sync_copy(nt_hbm, nt_v)
    pltpu.sync_copy(src_hbm.at[wid], src_v)
    pltpu.sync_copy(tgt_hbm.at[wid], tgt_v)
    plsc.subcore_barrier()

    @pl.loop(0, NCH * (CW // L))
    def _(i):
        j = i // (CW // L)
        m = i % (CW // L)
        sl = pl.ds(m * L, L)
        s16 = src_v[j, sl]
        t16 = tgt_v[j, sl]
        ts = plsc.load_gather(nt_v, [s16])
        seg_v[j, sl] = t16 * T + ts

    @pl.loop(0, NCH)
    def _(j):
        pltpu.sync_copy(ones_v, cnt_sh.at[seg_v.at[j]], add=True)

    plsc.subcore_barrier()
    pltpu.sync_copy(
        cnt_sh.at[pl.ds(sid * CNT_PW, CNT_PW)],
        out_hbm.at[cid, pl.ds(sid * CNT_PW, CNT_PW)],
    )


@functools.partial(
    pl.kernel,
    out_type=(
        jax.ShapeDtypeStruct((NW, NCH, CW), jnp.float32),
        jax.ShapeDtypeStruct((NW, NCH, CW), jnp.float32),
    ),
    mesh=_mesh,
    compiler_params=pltpu.CompilerParams(needs_layout_passes=False),
    scratch_types=[
        pltpu.VMEM((N,), jnp.int32),
        pltpu.VMEM((CNT,), jnp.float32),
        pltpu.VMEM((CNT_PW,), jnp.float32),
        pltpu.VMEM((L,), jnp.float32),
        pltpu.VMEM((L,), jnp.float32),
        pltpu.VMEM((NCH, CW), jnp.int32),
        pltpu.VMEM((NCH, CW), jnp.int32),
        pltpu.VMEM((NCH, CW), jnp.float32),
        pltpu.VMEM((NCH, CW), jnp.float32),
    ],
)
def _sc_w(src_hbm, tgt_hbm, nt_hbm, cnt_hbm, c1_hbm, c2_hbm, w1_hbm, w2_hbm,
          nt_v, cnt_v, cch_v, c1_v, c2_v, src_v, tgt_v, w1_v, w2_v):
    wid = _wid()
    pltpu.sync_copy(nt_hbm, nt_v)
    pltpu.sync_copy(cnt_hbm.at[0], cnt_v)
    pltpu.sync_copy(c1_hbm, c1_v)
    pltpu.sync_copy(c2_hbm, c2_v)
    pltpu.sync_copy(src_hbm.at[wid], src_v)
    pltpu.sync_copy(tgt_hbm.at[wid], tgt_v)

    for ch in range(NS):
        pltpu.sync_copy(cnt_hbm.at[1, pl.ds(ch * CNT_PW, CNT_PW)], cch_v)

        @pl.loop(0, CNT_PW // L)
        def _(i):
            sl_g = pl.ds(ch * CNT_PW + i * L, L)
            sl = pl.ds(i * L, L)
            cnt_v[sl_g] = cnt_v[sl_g] + cch_v[sl]

    @pl.loop(0, NCH * (CW // L))
    def _(i):
        j = i // (CW // L)
        m = i % (CW // L)
        sl = pl.ds(m * L, L)
        s16 = src_v[j, sl]
        t16 = tgt_v[j, sl]
        ts = plsc.load_gather(nt_v, [s16])
        tt = plsc.load_gather(nt_v, [t16])
        et = tt * T + ts
        cnt16 = plsc.load_gather(cnt_v, [t16 * T + ts])
        inv16 = 1.0 / jnp.maximum(cnt16, 1.0)
        w1_v[j, sl] = plsc.load_gather(c1_v, [et]) * inv16
        w2_v[j, sl] = plsc.load_gather(c2_v, [et]) * inv16

    pltpu.sync_copy(w1_v, w1_hbm.at[wid])
    pltpu.sync_copy(w2_v, w2_hbm.at[wid])


@functools.partial(
    pl.kernel,
    out_type=jax.ShapeDtypeStruct((2, NC, NPAD, DH), jnp.float32),
    mesh=_mesh,
    compiler_params=pltpu.CompilerParams(
        needs_layout_passes=False, use_tc_tiling_on_sc=False),
    scratch_types=[
        pltpu.VMEM((NCH, CW), jnp.int32),
        pltpu.VMEM((NCH, CW), jnp.int32),
        pltpu.VMEM((NCH, CW), jnp.float32),
        [pltpu.VMEM((CW, DH), jnp.float32) for _ in range(5)],
        pltpu.VMEM((ZR, DH), jnp.float32),
        [pltpu.SemaphoreType.DMA for _ in range(5)],
        [pltpu.SemaphoreType.DMA for _ in range(5)],
        pltpu.VMEM_SHARED((NPAD, DH), jnp.float32),
    ],
)
def _sc_msg(hlo_hbm, hhi_hbm, src_hbm, tgt_hbm, w_hbm, out_hbm,
            src_v, tgt_v, w_v, rows, zer_v, gsem, ssem, acc_sh):
    cid = lax.axis_index("c")
    sid = lax.axis_index("s")
    wid = _wid()
    NB = 5

    @pl.loop(0, ZR)
    def _(r):
        for m in range(DH // L):
            zer_v[r, pl.ds(m * L, L)] = jnp.zeros((L,), jnp.float32)

    pltpu.sync_copy(src_hbm.at[wid], src_v)
    pltpu.sync_copy(tgt_hbm.at[wid], tgt_v)
    pltpu.sync_copy(w_hbm.at[wid], w_v)

    for half, h_hbm in ((0, hlo_hbm), (1, hhi_hbm)):
        @pl.loop(0, RPW // ZR)
        def _(i):
            pltpu.sync_copy(zer_v, acc_sh.at[pl.ds(sid * RPW + i * ZR, ZR)])

        plsc.subcore_barrier()

        def _gather(j, b):
            pltpu.async_copy(h_hbm.at[src_v.at[j]], rows[b], gsem[b])

        def _wait_scatter(b):
            pltpu.make_async_copy(rows[b], acc_sh.at[tgt_v.at[0]],
                                  ssem[b]).wait()

        def _process(j, b):
            pltpu.make_async_copy(h_hbm.at[src_v.at[j]], rows[b],
                                  gsem[b]).wait()
            zero16 = lax.iota(jnp.int32, L) * 0
            jsplat = zero16 + j

            @pl.loop(0, CW, unroll=4)
            def _(k):
                wsv = plsc.load_gather(w_v, [jsplat, zero16 + k])
                for m in range(DH // L):
                    sl = pl.ds(m * L, L)
                    rows[b][k, sl] = rows[b][k, sl] * wsv

            pltpu.async_copy(rows[b], acc_sh.at[tgt_v.at[j]], ssem[b],
                             add=True)

        for j in range(NB - 1):
            _gather(j, j)
        _process(0, 0)
        _gather(NB - 1, NB - 1)
        for b in range(1, NB):
            _process(b, b)
            _wait_scatter((b + NB - 1) % NB)
            _gather(b + NB - 1, (b + NB - 1) % NB)

        @pl.loop(1, NCH // NB - 1)
        def _(kk):
            for b in range(NB):
                j = kk * NB + b
                _process(j, b)
                _wait_scatter((b + NB - 1) % NB)
                _gather(j + NB - 1, (b + NB - 1) % NB)

        j_last = (NCH // NB - 1) * NB
        _process(j_last, 0)
        _wait_scatter(NB - 1)
        _gather(j_last + NB - 1, NB - 1)
        for b in range(1, NB):
            _process(j_last + b, b)
        for b in range(NB):
            _wait_scatter(b)

        plsc.subcore_barrier()
        pltpu.sync_copy(
            acc_sh.at[pl.ds(sid * RPW, RPW)],
            out_hbm.at[half, cid, pl.ds(sid * RPW, RPW)],
        )
        plsc.subcore_barrier()


def _psum(p_ref):
    plo = p_ref[0, 0] + p_ref[0, 1]
    phi = p_ref[1, 0] + p_ref[1, 1]
    return jnp.concatenate([plo, phi], axis=1)


def _tc_prep_body(x_ref, w_ref, hlo_ref, hhi_ref, xr_ref):
    hx = jnp.dot(x_ref[...], w_ref[...], preferred_element_type=jnp.float32)
    hlo_ref[...] = hx[:, :DH]
    hhi_ref[...] = hx[:, DH:D]
    xr_ref[...] = hx[:, D:]


_tc_prep = pl.pallas_call(
    _tc_prep_body,
    grid=(N // BR,),
    in_specs=[
        pl.BlockSpec((BR, D), lambda i: (i, 0)),
        pl.BlockSpec((D, 2 * D), lambda i: (0, 0)),
    ],
    out_specs=[
        pl.BlockSpec((BR, DH), lambda i: (i, 0)),
        pl.BlockSpec((BR, DH), lambda i: (i, 0)),
        pl.BlockSpec((BR, D), lambda i: (i, 0)),
    ],
    out_shape=[
        jax.ShapeDtypeStruct((N, DH), jnp.float32),
        jax.ShapeDtypeStruct((N, DH), jnp.float32),
        jax.ShapeDtypeStruct((N, D), jnp.float32),
    ],
)


def _tc_fin1_body(p_ref, xr_ref, b_ref, w_ref, x1_ref, hlo_ref, hhi_ref, xr2_ref):
    s = _psum(p_ref) + xr_ref[...] + b_ref[...]
    x1 = jax.nn.sigmoid(s)
    x1_ref[...] = x1
    hx = jnp.dot(x1, w_ref[...], preferred_element_type=jnp.float32)
    hlo_ref[...] = hx[:, :DH]
    hhi_ref[...] = hx[:, DH:D]
    xr2_ref[...] = hx[:, D:]


_tc_fin1 = pl.pallas_call(
    _tc_fin1_body,
    grid=(N // BR,),
    in_specs=[
        pl.BlockSpec((2, NC, BR, DH), lambda i: (0, 0, i, 0)),
        pl.BlockSpec((BR, D), lambda i: (i, 0)),
        pl.BlockSpec((1, D), lambda i: (0, 0)),
        pl.BlockSpec((D, 2 * D), lambda i: (0, 0)),
    ],
    out_specs=[
        pl.BlockSpec((BR, D), lambda i: (i, 0)),
        pl.BlockSpec((BR, DH), lambda i: (i, 0)),
        pl.BlockSpec((BR, DH), lambda i: (i, 0)),
        pl.BlockSpec((BR, D), lambda i: (i, 0)),
    ],
    out_shape=[
        jax.ShapeDtypeStruct((N, D), jnp.float32),
        jax.ShapeDtypeStruct((N, DH), jnp.float32),
        jax.ShapeDtypeStruct((N, DH), jnp.float32),
        jax.ShapeDtypeStruct((N, D), jnp.float32),
    ],
)


def _tc_fin2_body(p_ref, xr_ref, b_ref, x1_ref, o_ref):
    o_ref[:, :D] = x1_ref[...]
    o_ref[:, D:] = jax.nn.sigmoid(_psum(p_ref) + xr_ref[...] + b_ref[...])


_tc_fin2 = pl.pallas_call(
    _tc_fin2_body,
    grid=(N // BR,),
    in_specs=[
        pl.BlockSpec((2, NC, BR, DH), lambda i: (0, 0, i, 0)),
        pl.BlockSpec((BR, D), lambda i: (i, 0)),
        pl.BlockSpec((1, D), lambda i: (0, 0)),
        pl.BlockSpec((BR, D), lambda i: (i, 0)),
    ],
    out_specs=pl.BlockSpec((BR, 2 * D), lambda i: (i, 0)),
    out_shape=jax.ShapeDtypeStruct((N, 2 * D), jnp.float32),
)


def kernel(x, edge_index, node_type, V1, comp1, root1, bias1,
           V2, comp2, root2, bias2):
    src = edge_index[0].astype(jnp.int32).reshape(NW, NCH, CW)
    tgt = edge_index[1].astype(jnp.int32).reshape(NW, NCH, CW)
    nt = node_type.astype(jnp.int32)
    w1cat = jnp.concatenate([V1[0], root1], axis=1)
    w2cat = jnp.concatenate([V2[0], root2], axis=1)
    c1 = comp1[:, 0]
    c2 = comp2[:, 0]

    cnt_part = _sc_counts(src, tgt, nt)
    w1, w2 = _sc_w(src, tgt, nt, cnt_part, c1, c2)
    h1lo, h1hi, xr1 = _tc_prep(x, w1cat)
    p1 = _sc_msg(h1lo, h1hi, src, tgt, w1)
    x1, h2lo, h2hi, xr2 = _tc_fin1(p1, xr1, bias1.reshape(1, D), w2cat)
    p2 = _sc_msg(h2lo, h2hi, src, tgt, w2)
    return _tc_fin2(p2, xr2, bias2.reshape(1, D), x1)


# speedup vs baseline: 1.0502x; 1.0502x over previous
import functools

import jax
import jax.numpy as jnp
from jax import lax
from jax.experimental import pallas as pl
from jax.experimental.pallas import tpu as pltpu
from jax.experimental.pallas import tpu_sc as plsc

N = 10000
E = 320000
D = 128
DH = D // 2
T = 4
L = 16
NC = 2
NS = 16
NW = NC * NS
EPW = E // NW
CW = 80
NCH = EPW // CW
NPAD = 10240
CNT = NPAD * T
CNT_PW = CNT // NS
RPW = NPAD // NS
ZR = 160
BR = 1000

_mesh = plsc.VectorSubcoreMesh(
    core_axis_name="c", subcore_axis_name="s", num_cores=NC, num_subcores=NS
)


def _wid():
    return lax.axis_index("s") * NC + lax.axis_index("c")


@functools.partial(
    pl.kernel,
    out_type=jax.ShapeDtypeStruct((NC, CNT), jnp.float32),
    mesh=_mesh,
    compiler_params=pltpu.CompilerParams(needs_layout_passes=False),
    scratch_types=[
        pltpu.VMEM((N,), jnp.int32),
        pltpu.VMEM((NCH, CW), jnp.int32),
        pltpu.VMEM((NCH, CW), jnp.int32),
        pltpu.VMEM((NCH, CW), jnp.int32),
        pltpu.VMEM((CW,), jnp.float32),
        pltpu.VMEM((CNT_PW,), jnp.float32),
        pltpu.VMEM_SHARED((CNT,), jnp.float32),
    ],
)
def _sc_counts(src_hbm, tgt_hbm, nt_hbm, out_hbm,
               nt_v, src_v, tgt_v, seg_v, ones_v, zero_v, cnt_sh):
    cid = lax.axis_index("c")
    sid = lax.axis_index("s")
    wid = _wid()

    @pl.loop(0, CW // L)
    def _(i):
        ones_v[pl.ds(i * L, L)] = jnp.ones((L,), jnp.float32)

    @pl.loop(0, CNT_PW // L)
    def _(i):
        zero_v[pl.ds(i * L, L)] = jnp.zeros((L,), jnp.float32)

    pltpu.sync_copy(zero_v, cnt_sh.at[pl.ds(sid * CNT_PW, CNT_PW)])
    pltpu.sync_copy(nt_hbm, nt_v)
    pltpu.sync_copy(src_hbm.at[wid], src_v)
    pltpu.sync_copy(tgt_hbm.at[wid], tgt_v)
    plsc.subcore_barrier()

    @pl.loop(0, NCH * (CW // L))
    def _(i):
        j = i // (CW // L)
        m = i % (CW // L)
        sl = pl.ds(m * L, L)
        s16 = src_v[j, sl]
        t16 = tgt_v[j, sl]
        ts = plsc.load_gather(nt_v, [s16])
        seg_v[j, sl] = t16 * T + ts

    @pl.loop(0, NCH)
    def _(j):
        pltpu.sync_copy(ones_v, cnt_sh.at[seg_v.at[j]], add=True)

    plsc.subcore_barrier()
    pltpu.sync_copy(
        cnt_sh.at[pl.ds(sid * CNT_PW, CNT_PW)],
        out_hbm.at[cid, pl.ds(sid * CNT_PW, CNT_PW)],
    )


@functools.partial(
    pl.kernel,
    out_type=(
        jax.ShapeDtypeStruct((NW, NCH, CW), jnp.float32),
        jax.ShapeDtypeStruct((NW, NCH, CW), jnp.float32),
    ),
    mesh=_mesh,
    compiler_params=pltpu.CompilerParams(needs_layout_passes=False),
    scratch_types=[
        pltpu.VMEM((N,), jnp.int32),
        pltpu.VMEM((CNT,), jnp.float32),
        pltpu.VMEM((L,), jnp.float32),
        pltpu.VMEM((L,), jnp.float32),
        pltpu.VMEM((NCH, CW), jnp.int32),
        pltpu.VMEM((NCH, CW), jnp.int32),
        pltpu.VMEM((NCH, CW), jnp.float32),
        pltpu.VMEM((NCH, CW), jnp.float32),
    ],
)
def _sc_w(src_hbm, tgt_hbm, nt_hbm, inv_hbm, c1_hbm, c2_hbm, w1_hbm, w2_hbm,
          nt_v, inv_v, c1_v, c2_v, src_v, tgt_v, w1_v, w2_v):
    wid = _wid()
    pltpu.sync_copy(nt_hbm, nt_v)
    pltpu.sync_copy(inv_hbm, inv_v)
    pltpu.sync_copy(c1_hbm, c1_v)
    pltpu.sync_copy(c2_hbm, c2_v)
    pltpu.sync_copy(src_hbm.at[wid], src_v)
    pltpu.sync_copy(tgt_hbm.at[wid], tgt_v)

    @pl.loop(0, NCH * (CW // L))
    def _(i):
        j = i // (CW // L)
        m = i % (CW // L)
        sl = pl.ds(m * L, L)
        s16 = src_v[j, sl]
        t16 = tgt_v[j, sl]
        ts = plsc.load_gather(nt_v, [s16])
        tt = plsc.load_gather(nt_v, [t16])
        et = tt * T + ts
        inv16 = plsc.load_gather(inv_v, [t16 * T + ts])
        w1_v[j, sl] = plsc.load_gather(c1_v, [et]) * inv16
        w2_v[j, sl] = plsc.load_gather(c2_v, [et]) * inv16

    pltpu.sync_copy(w1_v, w1_hbm.at[wid])
    pltpu.sync_copy(w2_v, w2_hbm.at[wid])


@functools.partial(
    pl.kernel,
    out_type=jax.ShapeDtypeStruct((2, NC, NPAD, DH), jnp.float32),
    mesh=_mesh,
    compiler_params=pltpu.CompilerParams(
        needs_layout_passes=False, use_tc_tiling_on_sc=False),
    scratch_types=[
        pltpu.VMEM((NCH, CW), jnp.int32),
        pltpu.VMEM((NCH, CW), jnp.int32),
        pltpu.VMEM((NCH, CW), jnp.float32),
        [pltpu.VMEM((CW, DH), jnp.float32) for _ in range(5)],
        pltpu.VMEM((ZR, DH), jnp.float32),
        [pltpu.SemaphoreType.DMA for _ in range(5)],
        [pltpu.SemaphoreType.DMA for _ in range(5)],
        pltpu.VMEM_SHARED((NPAD, DH), jnp.float32),
    ],
)
def _sc_msg(hlo_hbm, hhi_hbm, src_hbm, tgt_hbm, w_hbm, out_hbm,
            src_v, tgt_v, w_v, rows, zer_v, gsem, ssem, acc_sh):
    cid = lax.axis_index("c")
    sid = lax.axis_index("s")
    wid = _wid()
    NB = 5

    @pl.loop(0, ZR)
    def _(r):
        for m in range(DH // L):
            zer_v[r, pl.ds(m * L, L)] = jnp.zeros((L,), jnp.float32)

    pltpu.sync_copy(src_hbm.at[wid], src_v)
    pltpu.sync_copy(tgt_hbm.at[wid], tgt_v)
    pltpu.sync_copy(w_hbm.at[wid], w_v)

    for half, h_hbm in ((0, hlo_hbm), (1, hhi_hbm)):
        @pl.loop(0, RPW // ZR)
        def _(i):
            pltpu.sync_copy(zer_v, acc_sh.at[pl.ds(sid * RPW + i * ZR, ZR)])

        plsc.subcore_barrier()

        def _gather(j, b):
            pltpu.async_copy(h_hbm.at[src_v.at[j]], rows[b], gsem[b])

        def _wait_scatter(b):
            pltpu.make_async_copy(rows[b], acc_sh.at[tgt_v.at[0]],
                                  ssem[b]).wait()

        def _process(j, b):
            pltpu.make_async_copy(h_hbm.at[src_v.at[j]], rows[b],
                                  gsem[b]).wait()
            zero16 = lax.iota(jnp.int32, L) * 0
            jsplat = zero16 + j

            @pl.loop(0, CW, unroll=4)
            def _(k):
                wsv = plsc.load_gather(w_v, [jsplat, zero16 + k])
                for m in range(DH // L):
                    sl = pl.ds(m * L, L)
                    rows[b][k, sl] = rows[b][k, sl] * wsv

            pltpu.async_copy(rows[b], acc_sh.at[tgt_v.at[j]], ssem[b],
                             add=True)

        for j in range(NB - 1):
            _gather(j, j)
        _process(0, 0)
        _gather(NB - 1, NB - 1)
        for b in range(1, NB):
            _process(b, b)
            _wait_scatter((b + NB - 1) % NB)
            _gather(b + NB - 1, (b + NB - 1) % NB)

        @pl.loop(1, NCH // NB - 1)
        def _(kk):
            for b in range(NB):
                j = kk * NB + b
                _process(j, b)
                _wait_scatter((b + NB - 1) % NB)
                _gather(j + NB - 1, (b + NB - 1) % NB)

        j_last = (NCH // NB - 1) * NB
        _process(j_last, 0)
        _wait_scatter(NB - 1)
        _gather(j_last + NB - 1, NB - 1)
        for b in range(1, NB):
            _process(j_last + b, b)
        for b in range(NB):
            _wait_scatter(b)

        plsc.subcore_barrier()
        pltpu.sync_copy(
            acc_sh.at[pl.ds(sid * RPW, RPW)],
            out_hbm.at[half, cid, pl.ds(sid * RPW, RPW)],
        )
        plsc.subcore_barrier()


def _psum(p_ref):
    plo = p_ref[0, 0] + p_ref[0, 1]
    phi = p_ref[1, 0] + p_ref[1, 1]
    return jnp.concatenate([plo, phi], axis=1)


def _tc_prep_body(x_ref, w_ref, cnt_ref, hlo_ref, hhi_ref, xr_ref, inv_ref):
    hx = jnp.dot(x_ref[...], w_ref[...], preferred_element_type=jnp.float32)
    hlo_ref[...] = hx[:, :DH]
    hhi_ref[...] = hx[:, DH:D]
    xr_ref[...] = hx[:, D:]
    c = cnt_ref[0, :] + cnt_ref[1, :]
    inv_ref[...] = 1.0 / jnp.maximum(c, 1.0)


_tc_prep = pl.pallas_call(
    _tc_prep_body,
    grid=(N // BR,),
    in_specs=[
        pl.BlockSpec((BR, D), lambda i: (i, 0)),
        pl.BlockSpec((D, 2 * D), lambda i: (0, 0)),
        pl.BlockSpec((2, CNT // 10), lambda i: (0, i)),
    ],
    out_specs=[
        pl.BlockSpec((BR, DH), lambda i: (i, 0)),
        pl.BlockSpec((BR, DH), lambda i: (i, 0)),
        pl.BlockSpec((BR, D), lambda i: (i, 0)),
        pl.BlockSpec((CNT // 10,), lambda i: (i,)),
    ],
    out_shape=[
        jax.ShapeDtypeStruct((N, DH), jnp.float32),
        jax.ShapeDtypeStruct((N, DH), jnp.float32),
        jax.ShapeDtypeStruct((N, D), jnp.float32),
        jax.ShapeDtypeStruct((CNT,), jnp.float32),
    ],
)


def _tc_fin1_body(p_ref, xr_ref, b_ref, w_ref, x1_ref, hlo_ref, hhi_ref, xr2_ref):
    s = _psum(p_ref) + xr_ref[...] + b_ref[...]
    x1 = jax.nn.sigmoid(s)
    x1_ref[...] = x1
    hx = jnp.dot(x1, w_ref[...], preferred_element_type=jnp.float32)
    hlo_ref[...] = hx[:, :DH]
    hhi_ref[...] = hx[:, DH:D]
    xr2_ref[...] = hx[:, D:]


_tc_fin1 = pl.pallas_call(
    _tc_fin1_body,
    grid=(N // BR,),
    in_specs=[
        pl.BlockSpec((2, NC, BR, DH), lambda i: (0, 0, i, 0)),
        pl.BlockSpec((BR, D), lambda i: (i, 0)),
        pl.BlockSpec((1, D), lambda i: (0, 0)),
        pl.BlockSpec((D, 2 * D), lambda i: (0, 0)),
    ],
    out_specs=[
        pl.BlockSpec((BR, D), lambda i: (i, 0)),
        pl.BlockSpec((BR, DH), lambda i: (i, 0)),
        pl.BlockSpec((BR, DH), lambda i: (i, 0)),
        pl.BlockSpec((BR, D), lambda i: (i, 0)),
    ],
    out_shape=[
        jax.ShapeDtypeStruct((N, D), jnp.float32),
        jax.ShapeDtypeStruct((N, DH), jnp.float32),
        jax.ShapeDtypeStruct((N, DH), jnp.float32),
        jax.ShapeDtypeStruct((N, D), jnp.float32),
    ],
)


def _tc_fin2_body(p_ref, xr_ref, b_ref, x1_ref, o_ref):
    o_ref[:, :D] = x1_ref[...]
    o_ref[:, D:] = jax.nn.sigmoid(_psum(p_ref) + xr_ref[...] + b_ref[...])


_tc_fin2 = pl.pallas_call(
    _tc_fin2_body,
    grid=(N // BR,),
    in_specs=[
        pl.BlockSpec((2, NC, BR, DH), lambda i: (0, 0, i, 0)),
        pl.BlockSpec((BR, D), lambda i: (i, 0)),
        pl.BlockSpec((1, D), lambda i: (0, 0)),
        pl.BlockSpec((BR, D), lambda i: (i, 0)),
    ],
    out_specs=pl.BlockSpec((BR, 2 * D), lambda i: (i, 0)),
    out_shape=jax.ShapeDtypeStruct((N, 2 * D), jnp.float32),
)


def kernel(x, edge_index, node_type, V1, comp1, root1, bias1,
           V2, comp2, root2, bias2):
    src = edge_index[0].astype(jnp.int32).reshape(NW, NCH, CW)
    tgt = edge_index[1].astype(jnp.int32).reshape(NW, NCH, CW)
    nt = node_type.astype(jnp.int32)
    w1cat = jnp.concatenate([V1[0], root1], axis=1)
    w2cat = jnp.concatenate([V2[0], root2], axis=1)
    c1 = comp1[:, 0]
    c2 = comp2[:, 0]

    cnt_part = _sc_counts(src, tgt, nt)
    h1lo, h1hi, xr1, inv = _tc_prep(x, w1cat, cnt_part)
    w1, w2 = _sc_w(src, tgt, nt, inv, c1, c2)
    p1 = _sc_msg(h1lo, h1hi, src, tgt, w1)
    x1, h2lo, h2hi, xr2 = _tc_fin1(p1, xr1, bias1.reshape(1, D), w2cat)
    p2 = _sc_msg(h2lo, h2hi, src, tgt, w2)
    return _tc_fin2(p2, xr2, bias2.reshape(1, D), x1)
